# Initial kernel scaffold; baseline (speedup 1.0000x reference)
import jax
import jax.numpy as jnp
from jax.experimental import pallas as pl


def kernel(coords, hidden_state, cell_state, W_ih, W_hh, b_ih, b_hh):
    raise NotImplementedError("write your pallas kernel here")


# speedup vs baseline: 2.8029x; 2.8029x over previous
import functools
import jax
import jax.numpy as jnp
from jax import lax
from jax.experimental import pallas as pl
from jax.experimental.pallas import tpu as pltpu
from jax.experimental.pallas import tpu_sc as plsc

N = 65536
HS = 128
NG = 64
NCELL = NG * NG
X_MIN, X_MAX = -3.0, 3.0
Y_MIN, Y_MAX = -3.0, 3.0
DX = (X_MAX - X_MIN) / NG
DY = (Y_MAX - Y_MIN) / NG

_TC_B = 1024


def _lstm_tc_body(x_ref, h_ref, c_ref, wih_ref, whh_ref, b_ref,
                  hnew_ref, cnew_ref, cell_ref):
    x = x_ref[...]
    h = h_ref[...]
    c = c_ref[...]
    gates = (jnp.dot(x, wih_ref[...], preferred_element_type=jnp.float32)
             + jnp.dot(h, whh_ref[...], preferred_element_type=jnp.float32)
             + b_ref[...])
    i = jax.nn.sigmoid(gates[:, 0:HS])
    f = jax.nn.sigmoid(gates[:, HS:2 * HS])
    g = jnp.tanh(gates[:, 2 * HS:3 * HS])
    o = jax.nn.sigmoid(gates[:, 3 * HS:4 * HS])
    c_new = f * c + i * g
    hnew_ref[...] = o * jnp.tanh(c_new)
    cnew_ref[...] = c_new
    xc = jnp.clip(x[:, 0:1], X_MIN, X_MAX)
    yc = jnp.clip(x[:, 1:2], Y_MIN, Y_MAX)
    xi = jnp.clip(jnp.floor((xc - X_MIN) / DX).astype(jnp.int32), 0, NG - 1)
    yi = jnp.clip(jnp.floor((yc - Y_MIN) / DY).astype(jnp.int32), 0, NG - 1)
    cell_ref[...] = xi * NG + yi


def _lstm_tc(coords, h, c, wih_t, whh_t, b2, interpret=False):
    grid = (N // _TC_B,)
    return pl.pallas_call(
        _lstm_tc_body,
        grid=grid,
        in_specs=[
            pl.BlockSpec((_TC_B, 3), lambda i: (i, 0)),
            pl.BlockSpec((_TC_B, HS), lambda i: (i, 0)),
            pl.BlockSpec((_TC_B, HS), lambda i: (i, 0)),
            pl.BlockSpec((3, 4 * HS), lambda i: (0, 0)),
            pl.BlockSpec((HS, 4 * HS), lambda i: (0, 0)),
            pl.BlockSpec((1, 4 * HS), lambda i: (0, 0)),
        ],
        out_specs=[
            pl.BlockSpec((_TC_B, HS), lambda i: (i, 0)),
            pl.BlockSpec((_TC_B, HS), lambda i: (i, 0)),
            pl.BlockSpec((_TC_B, 1), lambda i: (i, 0)),
        ],
        out_shape=[
            jax.ShapeDtypeStruct((N, HS), jnp.float32),
            jax.ShapeDtypeStruct((N, HS), jnp.float32),
            jax.ShapeDtypeStruct((N, 1), jnp.int32),
        ],
        interpret=interpret,
    )(coords, h, c, wih_t, whh_t, b2)


_MESH = plsc.VectorSubcoreMesh(core_axis_name="c", subcore_axis_name="s")
_NROW = N // HS
_RPC = _NROW // 2 // 16
_RPG = _NROW // 32


def _sc_scatter(h_new, cell2d):
    @functools.partial(
        pl.kernel,
        out_type=[jax.ShapeDtypeStruct((NCELL, HS), jnp.float32),
                  jax.ShapeDtypeStruct((NCELL, HS), jnp.float32)],
        mesh=_MESH,
        scratch_types=[
            pltpu.VMEM_SHARED((NCELL, HS), jnp.float32),
            pltpu.VMEM((_RPC, HS), jnp.int32),
            pltpu.VMEM((HS, HS), jnp.float32),
        ],
    )
    def scatter_k(h_hbm, cell_hbm, t0_hbm, t1_hbm, tbl, idx_v, hbuf):
        c = lax.axis_index("c")
        s = lax.axis_index("s")
        @pl.loop(0, HS)
        def _(r):
            for cb in range(HS // 16):
                hbuf.at[r, pl.ds(cb * 16, 16)][...] = jnp.zeros((16,), jnp.float32)
        pltpu.sync_copy(hbuf, tbl.at[pl.ds(s * 256, HS)])
        pltpu.sync_copy(hbuf, tbl.at[pl.ds(s * 256 + HS, HS)])
        plsc.subcore_barrier()
        row0 = c * (_NROW // 2) + s * _RPC
        pltpu.sync_copy(cell_hbm.at[pl.ds(row0, _RPC)], idx_v)

        @pl.loop(0, _RPC)
        def _(j):
            pltpu.sync_copy(h_hbm.at[pl.ds((row0 + j) * HS, HS)], hbuf)
            pltpu.sync_copy(hbuf, tbl.at[idx_v.at[j]], add=True)

        plsc.subcore_barrier()

        @pl.when(c == 0)
        def _():
            pltpu.sync_copy(tbl.at[pl.ds(s * 256, 256)],
                            t0_hbm.at[pl.ds(s * 256, 256)])

        @pl.when(c == 1)
        def _():
            pltpu.sync_copy(tbl.at[pl.ds(s * 256, 256)],
                            t1_hbm.at[pl.ds(s * 256, 256)])

    return scatter_k(h_new, cell2d)


def _combine_body(a_ref, b_ref, o_ref):
    o_ref[...] = a_ref[...] + b_ref[...]


def _combine(t0, t1):
    return pl.pallas_call(
        _combine_body,
        grid=(4,),
        in_specs=[pl.BlockSpec((NCELL // 4, HS), lambda i: (i, 0)),
                  pl.BlockSpec((NCELL // 4, HS), lambda i: (i, 0))],
        out_specs=pl.BlockSpec((NCELL // 4, HS), lambda i: (i, 0)),
        out_shape=jax.ShapeDtypeStruct((NCELL, HS), jnp.float32),
    )(t0, t1)


def _sc_gather(table, cell2d):
    @functools.partial(
        pl.kernel,
        out_type=jax.ShapeDtypeStruct((N, HS), jnp.float32),
        mesh=_MESH,
        scratch_types=[
            pltpu.VMEM((_RPG, HS), jnp.int32),
            pltpu.VMEM((HS, HS), jnp.float32),
        ],
    )
    def gather_k(t_hbm, cell_hbm, o_hbm, idx_v, buf):
        c = lax.axis_index("c")
        s = lax.axis_index("s")
        row0 = (c * 16 + s) * _RPG
        pltpu.sync_copy(cell_hbm.at[pl.ds(row0, _RPG)], idx_v)

        @pl.loop(0, _RPG)
        def _(j):
            pltpu.sync_copy(t_hbm.at[idx_v.at[j]], buf)
            pltpu.sync_copy(buf, o_hbm.at[pl.ds((row0 + j) * HS, HS)])

    return gather_k(table, cell2d)


def kernel(coords, hidden_state, cell_state, W_ih, W_hh, b_ih, b_hh):
    wih_t = W_ih.T
    whh_t = W_hh.T
    b2 = (b_ih + b_hh)[None, :]
    h_new, c_new, cell = _lstm_tc(coords, hidden_state, cell_state,
                                  wih_t, whh_t, b2)
    cell2d = cell.reshape(_NROW, HS)
    t0, t1 = _sc_scatter(h_new, cell2d)
    table = _combine(t0, t1)
    h_social = _sc_gather(table, cell2d)
    return (h_social, c_new)


# speedup vs baseline: 3.1749x; 1.1327x over previous
import functools
import jax
import jax.numpy as jnp
from jax import lax
from jax.experimental import pallas as pl
from jax.experimental.pallas import tpu as pltpu
from jax.experimental.pallas import tpu_sc as plsc

N = 65536
HS = 128
NG = 64
NCELL = NG * NG
X_MIN, X_MAX = -3.0, 3.0
Y_MIN, Y_MAX = -3.0, 3.0
DX = (X_MAX - X_MIN) / NG
DY = (Y_MAX - Y_MIN) / NG

_TC_B = 1024
_NROW = N // HS
_RB = _TC_B // HS


def _lstm_tc_body(x_ref, h_ref, c_ref, xs_ref, ys_ref, wih_ref, whh_ref,
                  b_ref, hnew_ref, cnew_ref, cell_ref):
    x = x_ref[...]
    h = h_ref[...]
    c = c_ref[...]
    gates = (jnp.dot(x, wih_ref[...], preferred_element_type=jnp.float32)
             + jnp.dot(h, whh_ref[...], preferred_element_type=jnp.float32)
             + b_ref[...])
    i = jax.nn.sigmoid(gates[:, 0:HS])
    f = jax.nn.sigmoid(gates[:, HS:2 * HS])
    g = jnp.tanh(gates[:, 2 * HS:3 * HS])
    o = jax.nn.sigmoid(gates[:, 3 * HS:4 * HS])
    c_new = f * c + i * g
    hnew_ref[...] = o * jnp.tanh(c_new)
    cnew_ref[...] = c_new
    xc = jnp.clip(xs_ref[...], X_MIN, X_MAX)
    yc = jnp.clip(ys_ref[...], Y_MIN, Y_MAX)
    xi = jnp.clip(jnp.floor((xc - X_MIN) / DX).astype(jnp.int32), 0, NG - 1)
    yi = jnp.clip(jnp.floor((yc - Y_MIN) / DY).astype(jnp.int32), 0, NG - 1)
    cell_ref[...] = xi * NG + yi


def _lstm_tc(coords, h, c, xs2d, ys2d, wih_t, whh_t, b2, interpret=False):
    grid = (N // _TC_B,)
    return pl.pallas_call(
        _lstm_tc_body,
        grid=grid,
        in_specs=[
            pl.BlockSpec((_TC_B, 3), lambda i: (i, 0)),
            pl.BlockSpec((_TC_B, HS), lambda i: (i, 0)),
            pl.BlockSpec((_TC_B, HS), lambda i: (i, 0)),
            pl.BlockSpec((_RB, HS), lambda i: (i, 0)),
            pl.BlockSpec((_RB, HS), lambda i: (i, 0)),
            pl.BlockSpec((3, 4 * HS), lambda i: (0, 0)),
            pl.BlockSpec((HS, 4 * HS), lambda i: (0, 0)),
            pl.BlockSpec((1, 4 * HS), lambda i: (0, 0)),
        ],
        out_specs=[
            pl.BlockSpec((_TC_B, HS), lambda i: (i, 0)),
            pl.BlockSpec((_TC_B, HS), lambda i: (i, 0)),
            pl.BlockSpec((_RB, HS), lambda i: (i, 0)),
        ],
        out_shape=[
            jax.ShapeDtypeStruct((N, HS), jnp.float32),
            jax.ShapeDtypeStruct((N, HS), jnp.float32),
            jax.ShapeDtypeStruct((_NROW, HS), jnp.int32),
        ],
        interpret=interpret,
    )(coords, h, c, xs2d, ys2d, wih_t, whh_t, b2)


_MESH = plsc.VectorSubcoreMesh(core_axis_name="c", subcore_axis_name="s")
_RPC = _NROW // 2 // 16
_RPG = _NROW // 32


def _sc_scatter(h_new, cell2d):
    @functools.partial(
        pl.kernel,
        out_type=[jax.ShapeDtypeStruct((NCELL, HS), jnp.float32),
                  jax.ShapeDtypeStruct((NCELL, HS), jnp.float32)],
        mesh=_MESH,
        scratch_types=[
            pltpu.VMEM_SHARED((NCELL, HS), jnp.float32),
            pltpu.VMEM((_RPC, HS), jnp.int32),
            pltpu.VMEM((2, HS, HS), jnp.float32),
            pltpu.SemaphoreType.DMA,
            pltpu.SemaphoreType.DMA,
            pltpu.SemaphoreType.DMA,
            pltpu.SemaphoreType.DMA,
        ],
    )
    def scatter_k(h_hbm, cell_hbm, t0_hbm, t1_hbm, tbl, idx_v, hbuf,
                  sl0, sl1, ss0, ss1):
        c = lax.axis_index("c")
        s = lax.axis_index("s")
        @pl.loop(0, HS)
        def _(r):
            for cb in range(HS // 16):
                hbuf.at[0, r, pl.ds(cb * 16, 16)][...] = jnp.zeros(
                    (16,), jnp.float32)
        pltpu.sync_copy(hbuf.at[0], tbl.at[pl.ds(s * 256, HS)])
        pltpu.sync_copy(hbuf.at[0], tbl.at[pl.ds(s * 256 + HS, HS)])
        plsc.subcore_barrier()
        row0 = c * (_NROW // 2) + s * _RPC
        pltpu.sync_copy(cell_hbm.at[pl.ds(row0, _RPC)], idx_v)

        def load(j, slot, sem, start):
            args = (h_hbm.at[pl.ds((row0 + j) * HS, HS)], hbuf.at[slot], sem)
            if start:
                pltpu.async_copy(*args)
            else:
                pltpu.make_async_copy(*args).wait()

        def scat(j, slot, sem, start):
            if start:
                pltpu.async_copy(hbuf.at[slot], tbl.at[idx_v.at[j]], sem,
                                 add=True)
            else:
                pltpu.make_async_copy(hbuf.at[slot], tbl.at[idx_v.at[j]],
                                      sem).wait()

        load(0, 0, sl0, True)
        load(1, 1, sl1, True)

        @pl.loop(0, _RPC, step=2)
        def _(j):
            load(j, 0, sl0, False)
            scat(j, 0, ss0, True)
            load(j + 1, 1, sl1, False)
            scat(j + 1, 1, ss1, True)
            scat(j, 0, ss0, False)

            @pl.when(j + 2 < _RPC)
            def _():
                load(j + 2, 0, sl0, True)

            scat(j + 1, 1, ss1, False)

            @pl.when(j + 3 < _RPC)
            def _():
                load(j + 3, 1, sl1, True)

        plsc.subcore_barrier()

        @pl.when(c == 0)
        def _():
            pltpu.sync_copy(tbl.at[pl.ds(s * 256, 256)],
                            t0_hbm.at[pl.ds(s * 256, 256)])

        @pl.when(c == 1)
        def _():
            pltpu.sync_copy(tbl.at[pl.ds(s * 256, 256)],
                            t1_hbm.at[pl.ds(s * 256, 256)])

    return scatter_k(h_new, cell2d)


def _combine_body(a_ref, b_ref, o_ref):
    o_ref[...] = a_ref[...] + b_ref[...]


def _combine(t0, t1):
    return pl.pallas_call(
        _combine_body,
        grid=(4,),
        in_specs=[pl.BlockSpec((NCELL // 4, HS), lambda i: (i, 0)),
                  pl.BlockSpec((NCELL // 4, HS), lambda i: (i, 0))],
        out_specs=pl.BlockSpec((NCELL // 4, HS), lambda i: (i, 0)),
        out_shape=jax.ShapeDtypeStruct((NCELL, HS), jnp.float32),
    )(t0, t1)


def _sc_gather(table, cell2d):
    @functools.partial(
        pl.kernel,
        out_type=jax.ShapeDtypeStruct((N, HS), jnp.float32),
        mesh=_MESH,
        scratch_types=[
            pltpu.VMEM((_RPG, HS), jnp.int32),
            pltpu.VMEM((2, HS, HS), jnp.float32),
            pltpu.SemaphoreType.DMA,
            pltpu.SemaphoreType.DMA,
            pltpu.SemaphoreType.DMA,
            pltpu.SemaphoreType.DMA,
        ],
    )
    def gather_k(t_hbm, cell_hbm, o_hbm, idx_v, buf, sg0, sg1, sw0, sw1):
        c = lax.axis_index("c")
        s = lax.axis_index("s")
        row0 = (c * 16 + s) * _RPG
        pltpu.sync_copy(cell_hbm.at[pl.ds(row0, _RPG)], idx_v)

        def gath(j, slot, sem, start):
            args = (t_hbm.at[idx_v.at[j]], buf.at[slot], sem)
            if start:
                pltpu.async_copy(*args)
            else:
                pltpu.make_async_copy(*args).wait()

        def wout(j, slot, sem, start):
            args = (buf.at[slot], o_hbm.at[pl.ds((row0 + j) * HS, HS)], sem)
            if start:
                pltpu.async_copy(*args)
            else:
                pltpu.make_async_copy(*args).wait()

        gath(0, 0, sg0, True)
        gath(1, 1, sg1, True)

        @pl.loop(0, _RPG, step=2)
        def _(j):
            gath(j, 0, sg0, False)
            wout(j, 0, sw0, True)
            gath(j + 1, 1, sg1, False)
            wout(j + 1, 1, sw1, True)
            wout(j, 0, sw0, False)

            @pl.when(j + 2 < _RPG)
            def _():
                gath(j + 2, 0, sg0, True)

            wout(j + 1, 1, sw1, False)

            @pl.when(j + 3 < _RPG)
            def _():
                gath(j + 3, 1, sg1, True)

    return gather_k(table, cell2d)


def kernel(coords, hidden_state, cell_state, W_ih, W_hh, b_ih, b_hh):
    wih_t = W_ih.T
    whh_t = W_hh.T
    b2 = (b_ih + b_hh)[None, :]
    xs2d = coords[:, 0].reshape(_NROW, HS)
    ys2d = coords[:, 1].reshape(_NROW, HS)
    h_new, c_new, cell2d = _lstm_tc(coords, hidden_state, cell_state,
                                    xs2d, ys2d, wih_t, whh_t, b2)
    t0, t1 = _sc_scatter(h_new, cell2d)
    table = _combine(t0, t1)
    h_social = _sc_gather(table, cell2d)
    return (h_social, c_new)


# speedup vs baseline: 3.1953x; 1.0064x over previous
import functools
import jax
import jax.numpy as jnp
from jax import lax
from jax.experimental import pallas as pl
from jax.experimental.pallas import tpu as pltpu
from jax.experimental.pallas import tpu_sc as plsc

N = 65536
HS = 128
NG = 64
NCELL = NG * NG
X_MIN, X_MAX = -3.0, 3.0
Y_MIN, Y_MAX = -3.0, 3.0
DX = (X_MAX - X_MIN) / NG
DY = (Y_MAX - Y_MIN) / NG

_TC_B = 1024
_NROW = N // HS
_RB = _TC_B // HS


def _lstm_tc_body(x_ref, h_ref, c_ref, xs_ref, ys_ref, wih_ref, whh_ref,
                  b_ref, hnew_ref, cnew_ref, cell_ref):
    x = x_ref[...].astype(jnp.bfloat16)
    h = h_ref[...].astype(jnp.bfloat16)
    c = c_ref[...]
    gates = (jnp.dot(x, wih_ref[...], preferred_element_type=jnp.float32)
             + jnp.dot(h, whh_ref[...], preferred_element_type=jnp.float32)
             + b_ref[...])

    def sigmoid(z):
        return 0.5 * jnp.tanh(0.5 * z) + 0.5

    i = sigmoid(gates[:, 0:HS])
    f = sigmoid(gates[:, HS:2 * HS])
    g = jnp.tanh(gates[:, 2 * HS:3 * HS])
    o = sigmoid(gates[:, 3 * HS:4 * HS])
    c_new = f * c + i * g
    hnew_ref[...] = o * jnp.tanh(c_new)
    cnew_ref[...] = c_new
    xc = jnp.clip(xs_ref[...], X_MIN, X_MAX)
    yc = jnp.clip(ys_ref[...], Y_MIN, Y_MAX)
    xi = jnp.clip(jnp.floor((xc - X_MIN) / DX).astype(jnp.int32), 0, NG - 1)
    yi = jnp.clip(jnp.floor((yc - Y_MIN) / DY).astype(jnp.int32), 0, NG - 1)
    cell_ref[...] = xi * NG + yi


def _lstm_tc(coords, h, c, xs2d, ys2d, wih_t, whh_t, b2, interpret=False):
    grid = (N // _TC_B,)
    return pl.pallas_call(
        _lstm_tc_body,
        grid=grid,
        in_specs=[
            pl.BlockSpec((_TC_B, 3), lambda i: (i, 0)),
            pl.BlockSpec((_TC_B, HS), lambda i: (i, 0)),
            pl.BlockSpec((_TC_B, HS), lambda i: (i, 0)),
            pl.BlockSpec((_RB, HS), lambda i: (i, 0)),
            pl.BlockSpec((_RB, HS), lambda i: (i, 0)),
            pl.BlockSpec((3, 4 * HS), lambda i: (0, 0)),
            pl.BlockSpec((HS, 4 * HS), lambda i: (0, 0)),
            pl.BlockSpec((1, 4 * HS), lambda i: (0, 0)),
        ],
        out_specs=[
            pl.BlockSpec((_TC_B, HS), lambda i: (i, 0)),
            pl.BlockSpec((_TC_B, HS), lambda i: (i, 0)),
            pl.BlockSpec((_RB, HS), lambda i: (i, 0)),
        ],
        out_shape=[
            jax.ShapeDtypeStruct((N, HS), jnp.float32),
            jax.ShapeDtypeStruct((N, HS), jnp.float32),
            jax.ShapeDtypeStruct((_NROW, HS), jnp.int32),
        ],
        interpret=interpret,
    )(coords, h, c, xs2d, ys2d, wih_t, whh_t, b2)


@functools.cache
def _sc_mesh():
    return plsc.VectorSubcoreMesh(core_axis_name="c", subcore_axis_name="s",
                                  num_cores=2, num_subcores=16)
_RPC = _NROW // 2 // 16
_RPG = _NROW // 32


def _sc_scatter(h_new, cell2d):
    @functools.partial(
        pl.kernel,
        out_type=[jax.ShapeDtypeStruct((NCELL, HS), jnp.float32),
                  jax.ShapeDtypeStruct((NCELL, HS), jnp.float32)],
        mesh=_sc_mesh(),
        scratch_types=[
            pltpu.VMEM_SHARED((NCELL, HS), jnp.float32),
            pltpu.VMEM((_RPC, HS), jnp.int32),
            pltpu.VMEM((2, HS, HS), jnp.float32),
            pltpu.SemaphoreType.DMA,
            pltpu.SemaphoreType.DMA,
            pltpu.SemaphoreType.DMA,
            pltpu.SemaphoreType.DMA,
        ],
    )
    def scatter_k(h_hbm, cell_hbm, t0_hbm, t1_hbm, tbl, idx_v, hbuf,
                  sl0, sl1, ss0, ss1):
        c = lax.axis_index("c")
        s = lax.axis_index("s")
        @pl.loop(0, HS)
        def _(r):
            for cb in range(HS // 16):
                hbuf.at[0, r, pl.ds(cb * 16, 16)][...] = jnp.zeros(
                    (16,), jnp.float32)
        pltpu.sync_copy(hbuf.at[0], tbl.at[pl.ds(s * 256, HS)])
        pltpu.sync_copy(hbuf.at[0], tbl.at[pl.ds(s * 256 + HS, HS)])
        plsc.subcore_barrier()
        row0 = c * (_NROW // 2) + s * _RPC
        pltpu.sync_copy(cell_hbm.at[pl.ds(row0, _RPC)], idx_v)

        def load(j, slot, sem, start):
            args = (h_hbm.at[pl.ds((row0 + j) * HS, HS)], hbuf.at[slot], sem)
            if start:
                pltpu.async_copy(*args)
            else:
                pltpu.make_async_copy(*args).wait()

        def scat(j, slot, sem, start):
            if start:
                pltpu.async_copy(hbuf.at[slot], tbl.at[idx_v.at[j]], sem,
                                 add=True)
            else:
                pltpu.make_async_copy(hbuf.at[slot], tbl.at[idx_v.at[j]],
                                      sem).wait()

        load(0, 0, sl0, True)
        load(1, 1, sl1, True)

        @pl.loop(0, _RPC, step=2)
        def _(j):
            load(j, 0, sl0, False)
            scat(j, 0, ss0, True)
            load(j + 1, 1, sl1, False)
            scat(j + 1, 1, ss1, True)
            scat(j, 0, ss0, False)

            @pl.when(j + 2 < _RPC)
            def _():
                load(j + 2, 0, sl0, True)

            scat(j + 1, 1, ss1, False)

            @pl.when(j + 3 < _RPC)
            def _():
                load(j + 3, 1, sl1, True)

        plsc.subcore_barrier()

        @pl.when(c == 0)
        def _():
            pltpu.sync_copy(tbl.at[pl.ds(s * 256, 256)],
                            t0_hbm.at[pl.ds(s * 256, 256)])

        @pl.when(c == 1)
        def _():
            pltpu.sync_copy(tbl.at[pl.ds(s * 256, 256)],
                            t1_hbm.at[pl.ds(s * 256, 256)])

    return scatter_k(h_new, cell2d)


def _combine_body(a_ref, b_ref, o_ref):
    o_ref[...] = a_ref[...] + b_ref[...]


def _combine(t0, t1):
    return pl.pallas_call(
        _combine_body,
        grid=(4,),
        in_specs=[pl.BlockSpec((NCELL // 4, HS), lambda i: (i, 0)),
                  pl.BlockSpec((NCELL // 4, HS), lambda i: (i, 0))],
        out_specs=pl.BlockSpec((NCELL // 4, HS), lambda i: (i, 0)),
        out_shape=jax.ShapeDtypeStruct((NCELL, HS), jnp.float32),
    )(t0, t1)


def _sc_gather(table, cell2d):
    @functools.partial(
        pl.kernel,
        out_type=jax.ShapeDtypeStruct((N, HS), jnp.float32),
        mesh=_sc_mesh(),
        scratch_types=[
            pltpu.VMEM((_RPG, HS), jnp.int32),
            pltpu.VMEM((2, HS, HS), jnp.float32),
            pltpu.SemaphoreType.DMA,
            pltpu.SemaphoreType.DMA,
            pltpu.SemaphoreType.DMA,
            pltpu.SemaphoreType.DMA,
        ],
    )
    def gather_k(t_hbm, cell_hbm, o_hbm, idx_v, buf, sg0, sg1, sw0, sw1):
        c = lax.axis_index("c")
        s = lax.axis_index("s")
        row0 = (c * 16 + s) * _RPG
        pltpu.sync_copy(cell_hbm.at[pl.ds(row0, _RPG)], idx_v)

        def gath(j, slot, sem, start):
            args = (t_hbm.at[idx_v.at[j]], buf.at[slot], sem)
            if start:
                pltpu.async_copy(*args)
            else:
                pltpu.make_async_copy(*args).wait()

        def wout(j, slot, sem, start):
            args = (buf.at[slot], o_hbm.at[pl.ds((row0 + j) * HS, HS)], sem)
            if start:
                pltpu.async_copy(*args)
            else:
                pltpu.make_async_copy(*args).wait()

        gath(0, 0, sg0, True)
        gath(1, 1, sg1, True)

        @pl.loop(0, _RPG, step=2)
        def _(j):
            gath(j, 0, sg0, False)
            wout(j, 0, sw0, True)
            gath(j + 1, 1, sg1, False)
            wout(j + 1, 1, sw1, True)
            wout(j, 0, sw0, False)

            @pl.when(j + 2 < _RPG)
            def _():
                gath(j + 2, 0, sg0, True)

            wout(j + 1, 1, sw1, False)

            @pl.when(j + 3 < _RPG)
            def _():
                gath(j + 3, 1, sg1, True)

    return gather_k(table, cell2d)


def kernel(coords, hidden_state, cell_state, W_ih, W_hh, b_ih, b_hh):
    wih_t = W_ih.T.astype(jnp.bfloat16)
    whh_t = W_hh.T.astype(jnp.bfloat16)
    b2 = (b_ih + b_hh)[None, :]
    xs2d = coords[:, 0].reshape(_NROW, HS)
    ys2d = coords[:, 1].reshape(_NROW, HS)
    h_new, c_new, cell2d = _lstm_tc(coords, hidden_state, cell_state,
                                    xs2d, ys2d, wih_t, whh_t, b2)
    t0, t1 = _sc_scatter(h_new, cell2d)
    table = _combine(t0, t1)
    h_social = _sc_gather(table, cell2d)
    return (h_social, c_new)


# speedup vs baseline: 3.5921x; 1.1242x over previous
import functools
import jax
import jax.numpy as jnp
from jax import lax
from jax.experimental import pallas as pl
from jax.experimental.pallas import tpu as pltpu
from jax.experimental.pallas import tpu_sc as plsc

N = 65536
HS = 128
NG = 64
NCELL = NG * NG
X_MIN, X_MAX = -3.0, 3.0
Y_MIN, Y_MAX = -3.0, 3.0
DX = (X_MAX - X_MIN) / NG
DY = (Y_MAX - Y_MIN) / NG

_TC_B = 1024
_NROW = N // HS
_RB = _TC_B // HS


def _lstm_tc_body(x_ref, h_ref, c_ref, xs_ref, ys_ref, wih_ref, whh_ref,
                  b_ref, hnew_ref, cnew_ref, cell_ref):
    xt = x_ref[...]
    h = h_ref[...].astype(jnp.bfloat16)
    c = c_ref[...]
    gates = (lax.dot_general(xt, wih_ref[...], (((0,), (0,)), ((), ())),
                             preferred_element_type=jnp.float32)
             + jnp.dot(h, whh_ref[...], preferred_element_type=jnp.float32)
             + b_ref[...])

    def sigmoid(z):
        return 0.5 * jnp.tanh(0.5 * z) + 0.5

    i = sigmoid(gates[:, 0:HS])
    f = sigmoid(gates[:, HS:2 * HS])
    g = jnp.tanh(gates[:, 2 * HS:3 * HS])
    o = sigmoid(gates[:, 3 * HS:4 * HS])
    c_new = f * c + i * g
    hnew_ref[...] = o * jnp.tanh(c_new)
    cnew_ref[...] = c_new
    xc = jnp.clip(xs_ref[...], X_MIN, X_MAX)
    yc = jnp.clip(ys_ref[...], Y_MIN, Y_MAX)
    xi = jnp.clip(jnp.floor((xc - X_MIN) / DX).astype(jnp.int32), 0, NG - 1)
    yi = jnp.clip(jnp.floor((yc - Y_MIN) / DY).astype(jnp.int32), 0, NG - 1)
    cell_ref[...] = xi * NG + yi


def _lstm_tc(coords, h, c, xs2d, ys2d, wih_t, whh_t, b2, interpret=False):
    grid = (N // _TC_B,)
    return pl.pallas_call(
        _lstm_tc_body,
        grid=grid,
        in_specs=[
            pl.BlockSpec((3, _TC_B), lambda i: (0, i)),
            pl.BlockSpec((_TC_B, HS), lambda i: (i, 0)),
            pl.BlockSpec((_TC_B, HS), lambda i: (i, 0)),
            pl.BlockSpec((_RB, HS), lambda i: (i, 0)),
            pl.BlockSpec((_RB, HS), lambda i: (i, 0)),
            pl.BlockSpec((3, 4 * HS), lambda i: (0, 0)),
            pl.BlockSpec((HS, 4 * HS), lambda i: (0, 0)),
            pl.BlockSpec((1, 4 * HS), lambda i: (0, 0)),
        ],
        out_specs=[
            pl.BlockSpec((_TC_B, HS), lambda i: (i, 0)),
            pl.BlockSpec((_TC_B, HS), lambda i: (i, 0)),
            pl.BlockSpec((_RB, HS), lambda i: (i, 0)),
        ],
        out_shape=[
            jax.ShapeDtypeStruct((N, HS), jnp.float32),
            jax.ShapeDtypeStruct((N, HS), jnp.float32),
            jax.ShapeDtypeStruct((_NROW, HS), jnp.int32),
        ],
        interpret=interpret,
    )(coords, h, c, xs2d, ys2d, wih_t, whh_t, b2)


@functools.cache
def _sc_mesh():
    return plsc.VectorSubcoreMesh(core_axis_name="c", subcore_axis_name="s",
                                  num_cores=2, num_subcores=16)
_RPC = _NROW // 2 // 16
_RPG = _NROW // 32


def _sc_scatter(h_new, cell2d):
    @functools.partial(
        pl.kernel,
        out_type=[jax.ShapeDtypeStruct((NCELL, HS), jnp.float32),
                  jax.ShapeDtypeStruct((NCELL, HS), jnp.float32)],
        mesh=_sc_mesh(),
        scratch_types=[
            pltpu.VMEM_SHARED((NCELL, HS), jnp.float32),
            pltpu.VMEM((_RPC, HS), jnp.int32),
            pltpu.VMEM((2, HS, HS), jnp.float32),
            pltpu.SemaphoreType.DMA,
            pltpu.SemaphoreType.DMA,
            pltpu.SemaphoreType.DMA,
            pltpu.SemaphoreType.DMA,
        ],
    )
    def scatter_k(h_hbm, cell_hbm, t0_hbm, t1_hbm, tbl, idx_v, hbuf,
                  sl0, sl1, ss0, ss1):
        c = lax.axis_index("c")
        s = lax.axis_index("s")
        @pl.loop(0, HS)
        def _(r):
            for cb in range(HS // 16):
                hbuf.at[0, r, pl.ds(cb * 16, 16)][...] = jnp.zeros(
                    (16,), jnp.float32)
        pltpu.sync_copy(hbuf.at[0], tbl.at[pl.ds(s * 256, HS)])
        pltpu.sync_copy(hbuf.at[0], tbl.at[pl.ds(s * 256 + HS, HS)])
        plsc.subcore_barrier()
        row0 = c * (_NROW // 2) + s * _RPC
        pltpu.sync_copy(cell_hbm.at[pl.ds(row0, _RPC)], idx_v)

        def load(j, slot, sem, start):
            args = (h_hbm.at[pl.ds((row0 + j) * HS, HS)], hbuf.at[slot], sem)
            if start:
                pltpu.async_copy(*args)
            else:
                pltpu.make_async_copy(*args).wait()

        def scat(j, slot, sem, start):
            if start:
                pltpu.async_copy(hbuf.at[slot], tbl.at[idx_v.at[j]], sem,
                                 add=True)
            else:
                pltpu.make_async_copy(hbuf.at[slot], tbl.at[idx_v.at[j]],
                                      sem).wait()

        load(0, 0, sl0, True)
        load(1, 1, sl1, True)

        @pl.loop(0, _RPC, step=2)
        def _(j):
            load(j, 0, sl0, False)
            scat(j, 0, ss0, True)
            load(j + 1, 1, sl1, False)
            scat(j + 1, 1, ss1, True)
            scat(j, 0, ss0, False)

            @pl.when(j + 2 < _RPC)
            def _():
                load(j + 2, 0, sl0, True)

            scat(j + 1, 1, ss1, False)

            @pl.when(j + 3 < _RPC)
            def _():
                load(j + 3, 1, sl1, True)

        plsc.subcore_barrier()

        @pl.when(c == 0)
        def _():
            pltpu.sync_copy(tbl.at[pl.ds(s * 256, 256)],
                            t0_hbm.at[pl.ds(s * 256, 256)])

        @pl.when(c == 1)
        def _():
            pltpu.sync_copy(tbl.at[pl.ds(s * 256, 256)],
                            t1_hbm.at[pl.ds(s * 256, 256)])

    return scatter_k(h_new, cell2d)


def _combine_body(a_ref, b_ref, o_ref):
    o_ref[...] = a_ref[...] + b_ref[...]


def _combine(t0, t1):
    return pl.pallas_call(
        _combine_body,
        grid=(4,),
        in_specs=[pl.BlockSpec((NCELL // 4, HS), lambda i: (i, 0)),
                  pl.BlockSpec((NCELL // 4, HS), lambda i: (i, 0))],
        out_specs=pl.BlockSpec((NCELL // 4, HS), lambda i: (i, 0)),
        out_shape=jax.ShapeDtypeStruct((NCELL, HS), jnp.float32),
    )(t0, t1)


def _sc_gather(table, cell2d):
    @functools.partial(
        pl.kernel,
        out_type=jax.ShapeDtypeStruct((N, HS), jnp.float32),
        mesh=_sc_mesh(),
        scratch_types=[
            pltpu.VMEM((_RPG, HS), jnp.int32),
            pltpu.VMEM((2, HS, HS), jnp.float32),
            pltpu.SemaphoreType.DMA,
            pltpu.SemaphoreType.DMA,
            pltpu.SemaphoreType.DMA,
            pltpu.SemaphoreType.DMA,
        ],
    )
    def gather_k(t_hbm, cell_hbm, o_hbm, idx_v, buf, sg0, sg1, sw0, sw1):
        c = lax.axis_index("c")
        s = lax.axis_index("s")
        row0 = (c * 16 + s) * _RPG
        pltpu.sync_copy(cell_hbm.at[pl.ds(row0, _RPG)], idx_v)

        def gath(j, slot, sem, start):
            args = (t_hbm.at[idx_v.at[j]], buf.at[slot], sem)
            if start:
                pltpu.async_copy(*args)
            else:
                pltpu.make_async_copy(*args).wait()

        def wout(j, slot, sem, start):
            args = (buf.at[slot], o_hbm.at[pl.ds((row0 + j) * HS, HS)], sem)
            if start:
                pltpu.async_copy(*args)
            else:
                pltpu.make_async_copy(*args).wait()

        gath(0, 0, sg0, True)
        gath(1, 1, sg1, True)

        @pl.loop(0, _RPG, step=2)
        def _(j):
            gath(j, 0, sg0, False)
            wout(j, 0, sw0, True)
            gath(j + 1, 1, sg1, False)
            wout(j + 1, 1, sw1, True)
            wout(j, 0, sw0, False)

            @pl.when(j + 2 < _RPG)
            def _():
                gath(j + 2, 0, sg0, True)

            wout(j + 1, 1, sw1, False)

            @pl.when(j + 3 < _RPG)
            def _():
                gath(j + 3, 1, sg1, True)

    return gather_k(table, cell2d)


def kernel(coords, hidden_state, cell_state, W_ih, W_hh, b_ih, b_hh):
    wih_t = W_ih.T.astype(jnp.bfloat16)
    whh_t = W_hh.T.astype(jnp.bfloat16)
    b2 = (b_ih + b_hh)[None, :]
    coords_t = coords.T.astype(jnp.bfloat16)
    xs2d = coords[:, 0].reshape(_NROW, HS)
    ys2d = coords[:, 1].reshape(_NROW, HS)
    h_new, c_new, cell2d = _lstm_tc(coords_t, hidden_state, cell_state,
                                    xs2d, ys2d, wih_t, whh_t, b2)
    t0, t1 = _sc_scatter(h_new, cell2d)
    table = _combine(t0, t1)
    h_social = _sc_gather(table, cell2d)
    return (h_social, c_new)


# speedup vs baseline: 3.7790x; 1.0520x over previous
import functools
import jax
import jax.numpy as jnp
from jax import lax
from jax.experimental import pallas as pl
from jax.experimental.pallas import tpu as pltpu
from jax.experimental.pallas import tpu_sc as plsc

N = 65536
HS = 128
NG = 64
NCELL = NG * NG
X_MIN, X_MAX = -3.0, 3.0
Y_MIN, Y_MAX = -3.0, 3.0
DX = (X_MAX - X_MIN) / NG
DY = (Y_MAX - Y_MIN) / NG

_TC_B = 1024
_NROW = N // HS
_RB = _TC_B // HS


def _lstm_tc_body(x_ref, h_ref, c_ref, xs_ref, ys_ref, wih_ref, whh_ref,
                  b_ref, hnew_ref, cnew_ref, cell_ref):
    xt = x_ref[...]
    h = h_ref[...].astype(jnp.bfloat16)
    c = c_ref[...]
    gates = (lax.dot_general(xt, wih_ref[...], (((0,), (0,)), ((), ())),
                             preferred_element_type=jnp.float32)
             + jnp.dot(h, whh_ref[...], preferred_element_type=jnp.float32)
             + b_ref[...])

    def sigmoid(z):
        return 0.5 * jnp.tanh(0.5 * z) + 0.5

    i = sigmoid(gates[:, 0:HS])
    f = sigmoid(gates[:, HS:2 * HS])
    g = jnp.tanh(gates[:, 2 * HS:3 * HS])
    o = sigmoid(gates[:, 3 * HS:4 * HS])
    c_new = f * c + i * g
    hnew_ref[...] = o * jnp.tanh(c_new)
    cnew_ref[...] = c_new
    xc = jnp.clip(xs_ref[...], X_MIN, X_MAX)
    yc = jnp.clip(ys_ref[...], Y_MIN, Y_MAX)
    xi = jnp.clip(jnp.floor((xc - X_MIN) / DX).astype(jnp.int32), 0, NG - 1)
    yi = jnp.clip(jnp.floor((yc - Y_MIN) / DY).astype(jnp.int32), 0, NG - 1)
    cell_ref[...] = xi * NG + yi


_NCHUNK = 2
_CB = N // _NCHUNK // _TC_B
_CROW = _NROW // _NCHUNK


def _lstm_tc_body2(x_ref, h_ref, c_ref, xs_ref, ys_ref, wih_ref, whh_ref,
                   b_ref, cdest_ref, hnew_ref, cnew_ref, cell_ref):
    del cdest_ref
    _lstm_tc_body(x_ref, h_ref, c_ref, xs_ref, ys_ref, wih_ref, whh_ref,
                  b_ref, hnew_ref, cnew_ref, cell_ref)


def _lstm_tc(k, coords_t, h, c, xs2d, ys2d, wih_t, whh_t, b2, c_donate=None,
             interpret=False):
    in_specs = [
        pl.BlockSpec((3, _TC_B), lambda i: (0, i + k * _CB)),
        pl.BlockSpec((_TC_B, HS), lambda i: (i + k * _CB, 0)),
        pl.BlockSpec((_TC_B, HS), lambda i: (i + k * _CB, 0)),
        pl.BlockSpec((_RB, HS), lambda i: (i + k * _CB, 0)),
        pl.BlockSpec((_RB, HS), lambda i: (i + k * _CB, 0)),
        pl.BlockSpec((3, 4 * HS), lambda i: (0, 0)),
        pl.BlockSpec((HS, 4 * HS), lambda i: (0, 0)),
        pl.BlockSpec((1, 4 * HS), lambda i: (0, 0)),
    ]
    args = [coords_t, h, c, xs2d, ys2d, wih_t, whh_t, b2]
    if c_donate is None:
        body = _lstm_tc_body
        aliases = {}
    else:
        body = _lstm_tc_body2
        in_specs = in_specs + [pl.BlockSpec((8, HS), lambda i: (0, 0))]
        args = args + [c_donate]
        aliases = {8: 1}
    return pl.pallas_call(
        body,
        grid=(_CB,),
        in_specs=in_specs,
        out_specs=[
            pl.BlockSpec((_TC_B, HS), lambda i: (i, 0)),
            pl.BlockSpec((_TC_B, HS), lambda i: (i + k * _CB, 0)),
            pl.BlockSpec((_RB, HS), lambda i: (i, 0)),
        ],
        out_shape=[
            jax.ShapeDtypeStruct((N // _NCHUNK, HS), jnp.float32),
            jax.ShapeDtypeStruct((N, HS), jnp.float32),
            jax.ShapeDtypeStruct((_CROW, HS), jnp.int32),
        ],
        input_output_aliases=aliases,
        interpret=interpret,
    )(*args)


@functools.cache
def _sc_mesh():
    return plsc.VectorSubcoreMesh(core_axis_name="c", subcore_axis_name="s",
                                  num_cores=2, num_subcores=16)
_RPC = _CROW // 2 // 16
_RPG = _NROW // 32


def _sc_scatter(h_new, cell2d):
    @functools.partial(
        pl.kernel,
        out_type=[jax.ShapeDtypeStruct((NCELL, HS), jnp.float32),
                  jax.ShapeDtypeStruct((NCELL, HS), jnp.float32)],
        mesh=_sc_mesh(),
        scratch_types=[
            pltpu.VMEM_SHARED((NCELL, HS), jnp.float32),
            pltpu.VMEM((_RPC, HS), jnp.int32),
            pltpu.VMEM((2, HS, HS), jnp.float32),
            pltpu.SemaphoreType.DMA,
            pltpu.SemaphoreType.DMA,
            pltpu.SemaphoreType.DMA,
            pltpu.SemaphoreType.DMA,
        ],
    )
    def scatter_k(h_hbm, cell_hbm, t0_hbm, t1_hbm, tbl, idx_v, hbuf,
                  sl0, sl1, ss0, ss1):
        c = lax.axis_index("c")
        s = lax.axis_index("s")
        @pl.loop(0, HS)
        def _(r):
            for cb in range(HS // 16):
                hbuf.at[0, r, pl.ds(cb * 16, 16)][...] = jnp.zeros(
                    (16,), jnp.float32)
        pltpu.sync_copy(hbuf.at[0], tbl.at[pl.ds(s * 256, HS)])
        pltpu.sync_copy(hbuf.at[0], tbl.at[pl.ds(s * 256 + HS, HS)])
        plsc.subcore_barrier()
        row0 = c * (_CROW // 2) + s * _RPC
        pltpu.sync_copy(cell_hbm.at[pl.ds(row0, _RPC)], idx_v)

        def load(j, slot, sem, start):
            args = (h_hbm.at[pl.ds((row0 + j) * HS, HS)], hbuf.at[slot], sem)
            if start:
                pltpu.async_copy(*args)
            else:
                pltpu.make_async_copy(*args).wait()

        def scat(j, slot, sem, start):
            if start:
                pltpu.async_copy(hbuf.at[slot], tbl.at[idx_v.at[j]], sem,
                                 add=True)
            else:
                pltpu.make_async_copy(hbuf.at[slot], tbl.at[idx_v.at[j]],
                                      sem).wait()

        load(0, 0, sl0, True)
        load(1, 1, sl1, True)

        @pl.loop(0, _RPC, step=2)
        def _(j):
            load(j, 0, sl0, False)
            scat(j, 0, ss0, True)
            load(j + 1, 1, sl1, False)
            scat(j + 1, 1, ss1, True)
            scat(j, 0, ss0, False)

            @pl.when(j + 2 < _RPC)
            def _():
                load(j + 2, 0, sl0, True)

            scat(j + 1, 1, ss1, False)

            @pl.when(j + 3 < _RPC)
            def _():
                load(j + 3, 1, sl1, True)

        plsc.subcore_barrier()

        @pl.when(c == 0)
        def _():
            pltpu.sync_copy(tbl.at[pl.ds(s * 256, 256)],
                            t0_hbm.at[pl.ds(s * 256, 256)])

        @pl.when(c == 1)
        def _():
            pltpu.sync_copy(tbl.at[pl.ds(s * 256, 256)],
                            t1_hbm.at[pl.ds(s * 256, 256)])

    return scatter_k(h_new, cell2d)


def _combine_body(a_ref, b_ref, c_ref, d_ref, o_ref):
    o_ref[...] = (a_ref[...] + b_ref[...]) + (c_ref[...] + d_ref[...])


def _combine(t0, t1, t2, t3):
    spec = pl.BlockSpec((NCELL // 4, HS), lambda i: (i, 0))
    return pl.pallas_call(
        _combine_body,
        grid=(4,),
        in_specs=[spec] * 4,
        out_specs=spec,
        out_shape=jax.ShapeDtypeStruct((NCELL, HS), jnp.float32),
    )(t0, t1, t2, t3)


def _sc_gather(table, cell2d):
    @functools.partial(
        pl.kernel,
        out_type=jax.ShapeDtypeStruct((N, HS), jnp.float32),
        mesh=_sc_mesh(),
        scratch_types=[
            pltpu.VMEM((_RPG, HS), jnp.int32),
            pltpu.VMEM((2, HS, HS), jnp.float32),
            pltpu.SemaphoreType.DMA,
            pltpu.SemaphoreType.DMA,
            pltpu.SemaphoreType.DMA,
            pltpu.SemaphoreType.DMA,
        ],
    )
    def gather_k(t_hbm, cell_hbm, o_hbm, idx_v, buf, sg0, sg1, sw0, sw1):
        c = lax.axis_index("c")
        s = lax.axis_index("s")
        row0 = (c * 16 + s) * _RPG
        pltpu.sync_copy(cell_hbm.at[pl.ds(row0, _RPG)], idx_v)

        def gath(j, slot, sem, start):
            args = (t_hbm.at[idx_v.at[j]], buf.at[slot], sem)
            if start:
                pltpu.async_copy(*args)
            else:
                pltpu.make_async_copy(*args).wait()

        def wout(j, slot, sem, start):
            args = (buf.at[slot], o_hbm.at[pl.ds((row0 + j) * HS, HS)], sem)
            if start:
                pltpu.async_copy(*args)
            else:
                pltpu.make_async_copy(*args).wait()

        gath(0, 0, sg0, True)
        gath(1, 1, sg1, True)

        @pl.loop(0, _RPG, step=2)
        def _(j):
            gath(j, 0, sg0, False)
            wout(j, 0, sw0, True)
            gath(j + 1, 1, sg1, False)
            wout(j + 1, 1, sw1, True)
            wout(j, 0, sw0, False)

            @pl.when(j + 2 < _RPG)
            def _():
                gath(j + 2, 0, sg0, True)

            wout(j + 1, 1, sw1, False)

            @pl.when(j + 3 < _RPG)
            def _():
                gath(j + 3, 1, sg1, True)

    return gather_k(table, cell2d)


def kernel(coords, hidden_state, cell_state, W_ih, W_hh, b_ih, b_hh):
    wih_t = W_ih.T.astype(jnp.bfloat16)
    whh_t = W_hh.T.astype(jnp.bfloat16)
    b2 = (b_ih + b_hh)[None, :]
    coords_t = coords.T.astype(jnp.bfloat16)
    xs2d = coords[:, 0].reshape(_NROW, HS)
    ys2d = coords[:, 1].reshape(_NROW, HS)
    h0, c_v1, cd0 = _lstm_tc(0, coords_t, hidden_state, cell_state,
                             xs2d, ys2d, wih_t, whh_t, b2)
    h1, c_new, cd1 = _lstm_tc(1, coords_t, hidden_state, cell_state,
                              xs2d, ys2d, wih_t, whh_t, b2, c_donate=c_v1)
    t00, t01 = _sc_scatter(h0, cd0)
    t10, t11 = _sc_scatter(h1, cd1)
    table = _combine(t00, t01, t10, t11)
    cell2d = jnp.concatenate([cd0, cd1], axis=0)
    h_social = _sc_gather(table, cell2d)
    return (h_social, c_new)


# speedup vs baseline: 3.8915x; 1.0298x over previous
import functools
import jax
import jax.numpy as jnp
from jax import lax
from jax.experimental import pallas as pl
from jax.experimental.pallas import tpu as pltpu
from jax.experimental.pallas import tpu_sc as plsc

N = 65536
HS = 128
NG = 64
NCELL = NG * NG
X_MIN, X_MAX = -3.0, 3.0
Y_MIN, Y_MAX = -3.0, 3.0
DX = (X_MAX - X_MIN) / NG
DY = (Y_MAX - Y_MIN) / NG

_TC_B = 1024
_NROW = N // HS
_RB = _TC_B // HS


def _lstm_tc_body(x_ref, h_ref, c_ref, xs_ref, ys_ref, wih_ref, whh_ref,
                  b_ref, hnew_ref, cnew_ref, cell_ref):
    xt = x_ref[...]
    h = h_ref[...].astype(jnp.bfloat16)
    c = c_ref[...]
    gates = (lax.dot_general(xt, wih_ref[...], (((0,), (0,)), ((), ())),
                             preferred_element_type=jnp.float32)
             + jnp.dot(h, whh_ref[...], preferred_element_type=jnp.float32)
             + b_ref[...])

    def sigmoid(z):
        return 0.5 * jnp.tanh(0.5 * z) + 0.5

    i = sigmoid(gates[:, 0:HS])
    f = sigmoid(gates[:, HS:2 * HS])
    g = jnp.tanh(gates[:, 2 * HS:3 * HS])
    o = sigmoid(gates[:, 3 * HS:4 * HS])
    c_new = f * c + i * g
    hnew_ref[...] = o * jnp.tanh(c_new)
    cnew_ref[...] = c_new
    xc = jnp.clip(xs_ref[...], X_MIN, X_MAX)
    yc = jnp.clip(ys_ref[...], Y_MIN, Y_MAX)
    xi = jnp.clip(jnp.floor((xc - X_MIN) / DX).astype(jnp.int32), 0, NG - 1)
    yi = jnp.clip(jnp.floor((yc - Y_MIN) / DY).astype(jnp.int32), 0, NG - 1)
    cell_ref[...] = xi * NG + yi


_NCHUNK = 2
_CB = N // _NCHUNK // _TC_B
_CROW = _NROW // _NCHUNK


def _lstm_tc_body2(x_ref, h_ref, c_ref, xs_ref, ys_ref, wih_ref, whh_ref,
                   b_ref, cdest_ref, hnew_ref, cnew_ref, cell_ref):
    del cdest_ref
    _lstm_tc_body(x_ref, h_ref, c_ref, xs_ref, ys_ref, wih_ref, whh_ref,
                  b_ref, hnew_ref, cnew_ref, cell_ref)


def _lstm_tc(k, coords_t, h, c, xs2d, ys2d, wih_t, whh_t, b2, c_donate=None,
             interpret=False):
    in_specs = [
        pl.BlockSpec((3, _TC_B), lambda i: (0, i + k * _CB)),
        pl.BlockSpec((_TC_B, HS), lambda i: (i + k * _CB, 0)),
        pl.BlockSpec((_TC_B, HS), lambda i: (i + k * _CB, 0)),
        pl.BlockSpec((_RB, HS), lambda i: (i + k * _CB, 0)),
        pl.BlockSpec((_RB, HS), lambda i: (i + k * _CB, 0)),
        pl.BlockSpec((3, 4 * HS), lambda i: (0, 0)),
        pl.BlockSpec((HS, 4 * HS), lambda i: (0, 0)),
        pl.BlockSpec((1, 4 * HS), lambda i: (0, 0)),
    ]
    args = [coords_t, h, c, xs2d, ys2d, wih_t, whh_t, b2]
    if c_donate is None:
        body = _lstm_tc_body
        aliases = {}
    else:
        body = _lstm_tc_body2
        in_specs = in_specs + [pl.BlockSpec((8, HS), lambda i: (0, 0))]
        args = args + [c_donate]
        aliases = {8: 1}
    return pl.pallas_call(
        body,
        grid=(_CB,),
        in_specs=in_specs,
        out_specs=[
            pl.BlockSpec((_TC_B, HS), lambda i: (i, 0)),
            pl.BlockSpec((_TC_B, HS), lambda i: (i + k * _CB, 0)),
            pl.BlockSpec((_RB, HS), lambda i: (i, 0)),
        ],
        out_shape=[
            jax.ShapeDtypeStruct((N // _NCHUNK, HS), jnp.float32),
            jax.ShapeDtypeStruct((N, HS), jnp.float32),
            jax.ShapeDtypeStruct((_CROW, HS), jnp.int32),
        ],
        input_output_aliases=aliases,
        interpret=interpret,
    )(*args)


@functools.cache
def _sc_mesh():
    return plsc.VectorSubcoreMesh(core_axis_name="c", subcore_axis_name="s",
                                  num_cores=2, num_subcores=16)
_RPC = _CROW // 2 // 16
_RPG = _NROW // 32


def _sc_scatter(h_new, cell2d):
    @functools.partial(
        pl.kernel,
        out_type=[jax.ShapeDtypeStruct((NCELL, HS), jnp.float32),
                  jax.ShapeDtypeStruct((NCELL, HS), jnp.float32)],
        mesh=_sc_mesh(),
        scratch_types=[
            pltpu.VMEM_SHARED((NCELL, HS), jnp.float32),
            pltpu.VMEM((_RPC, HS), jnp.int32),
            pltpu.VMEM((2, HS, HS), jnp.float32),
            pltpu.SemaphoreType.DMA,
            pltpu.SemaphoreType.DMA,
            pltpu.SemaphoreType.DMA,
            pltpu.SemaphoreType.DMA,
        ],
    )
    def scatter_k(h_hbm, cell_hbm, t0_hbm, t1_hbm, tbl, idx_v, hbuf,
                  sl0, sl1, ss0, ss1):
        c = lax.axis_index("c")
        s = lax.axis_index("s")
        @pl.loop(0, HS)
        def _(r):
            for cb in range(HS // 16):
                hbuf.at[0, r, pl.ds(cb * 16, 16)][...] = jnp.zeros(
                    (16,), jnp.float32)
        pltpu.sync_copy(hbuf.at[0], tbl.at[pl.ds(s * 256, HS)])
        pltpu.sync_copy(hbuf.at[0], tbl.at[pl.ds(s * 256 + HS, HS)])
        plsc.subcore_barrier()
        row0 = c * (_CROW // 2) + s * _RPC
        pltpu.sync_copy(cell_hbm.at[pl.ds(row0, _RPC)], idx_v)

        def load(j, slot, sem, start):
            args = (h_hbm.at[pl.ds((row0 + j) * HS, HS)], hbuf.at[slot], sem)
            if start:
                pltpu.async_copy(*args)
            else:
                pltpu.make_async_copy(*args).wait()

        def scat(j, slot, sem, start):
            if start:
                pltpu.async_copy(hbuf.at[slot], tbl.at[idx_v.at[j]], sem,
                                 add=True)
            else:
                pltpu.make_async_copy(hbuf.at[slot], tbl.at[idx_v.at[j]],
                                      sem).wait()

        load(0, 0, sl0, True)
        load(1, 1, sl1, True)

        @pl.loop(0, _RPC, step=2)
        def _(j):
            load(j, 0, sl0, False)
            scat(j, 0, ss0, True)
            load(j + 1, 1, sl1, False)
            scat(j + 1, 1, ss1, True)
            scat(j, 0, ss0, False)

            @pl.when(j + 2 < _RPC)
            def _():
                load(j + 2, 0, sl0, True)

            scat(j + 1, 1, ss1, False)

            @pl.when(j + 3 < _RPC)
            def _():
                load(j + 3, 1, sl1, True)

        plsc.subcore_barrier()

        @pl.when(c == 0)
        def _():
            pltpu.sync_copy(tbl.at[pl.ds(s * 256, 256)],
                            t0_hbm.at[pl.ds(s * 256, 256)])

        @pl.when(c == 1)
        def _():
            pltpu.sync_copy(tbl.at[pl.ds(s * 256, 256)],
                            t1_hbm.at[pl.ds(s * 256, 256)])

    return scatter_k(h_new, cell2d)


def _combine_body(a_ref, b_ref, c_ref, d_ref, o_ref):
    o_ref[...] = (a_ref[...] + b_ref[...]) + (c_ref[...] + d_ref[...])


def _combine(t0, t1, t2, t3):
    spec = pl.BlockSpec((NCELL // 4, HS), lambda i: (i, 0))
    return pl.pallas_call(
        _combine_body,
        grid=(4,),
        in_specs=[spec] * 4,
        out_specs=spec,
        out_shape=jax.ShapeDtypeStruct((NCELL, HS), jnp.float32),
    )(t0, t1, t2, t3)


def _sc_gather(tabs, cell2d, iota2d):
    @functools.partial(
        pl.kernel,
        out_type=jax.ShapeDtypeStruct((N, HS), jnp.float32),
        mesh=_sc_mesh(),
        scratch_types=[
            pltpu.VMEM_SHARED((NCELL, HS), jnp.float32),
            pltpu.VMEM((2, HS), jnp.int32),
            pltpu.VMEM((2, HS, HS), jnp.float32),
            pltpu.VMEM((_RPG, HS), jnp.int32),
            pltpu.VMEM((2, HS, HS), jnp.float32),
            pltpu.SemaphoreType.DMA,
            pltpu.SemaphoreType.DMA,
            pltpu.SemaphoreType.DMA,
            pltpu.SemaphoreType.DMA,
        ],
    )
    def gather_k(ta_hbm, tb_hbm, tc_hbm, td_hbm, cell_hbm, iota_hbm, o_hbm,
                 tbl, iv, sbuf, idx_v, buf, sg0, sg1, sw0, sw1):
        c = lax.axis_index("c")
        s = lax.axis_index("s")
        pltpu.sync_copy(iota_hbm.at[pl.ds(2 * s, 2)], iv)
        pltpu.sync_copy(ta_hbm.at[pl.ds(s * 256, 256)],
                        tbl.at[pl.ds(s * 256, 256)])
        for h in range(2):
            base = s * 256 + h * HS
            pltpu.sync_copy(tb_hbm.at[pl.ds(base, HS)], sbuf.at[0])
            pltpu.async_copy(sbuf.at[0], tbl.at[iv.at[h]], sg0, add=True)
            pltpu.sync_copy(tc_hbm.at[pl.ds(base, HS)], sbuf.at[1])
            pltpu.async_copy(sbuf.at[1], tbl.at[iv.at[h]], sg1, add=True)
            pltpu.make_async_copy(sbuf.at[0], tbl.at[iv.at[h]], sg0).wait()
            pltpu.sync_copy(td_hbm.at[pl.ds(base, HS)], sbuf.at[0])
            pltpu.async_copy(sbuf.at[0], tbl.at[iv.at[h]], sg0, add=True)
            pltpu.make_async_copy(sbuf.at[1], tbl.at[iv.at[h]], sg1).wait()
            pltpu.make_async_copy(sbuf.at[0], tbl.at[iv.at[h]], sg0).wait()
        plsc.subcore_barrier()
        row0 = (c * 16 + s) * _RPG
        pltpu.sync_copy(cell_hbm.at[pl.ds(row0, _RPG)], idx_v)

        def gath(j, slot, sem, start):
            args = (tbl.at[idx_v.at[j]], buf.at[slot], sem)
            if start:
                pltpu.async_copy(*args)
            else:
                pltpu.make_async_copy(*args).wait()

        def wout(j, slot, sem, start):
            args = (buf.at[slot], o_hbm.at[pl.ds((row0 + j) * HS, HS)], sem)
            if start:
                pltpu.async_copy(*args)
            else:
                pltpu.make_async_copy(*args).wait()

        gath(0, 0, sg0, True)
        gath(1, 1, sg1, True)

        @pl.loop(0, _RPG, step=2)
        def _(j):
            gath(j, 0, sg0, False)
            wout(j, 0, sw0, True)
            gath(j + 1, 1, sg1, False)
            wout(j + 1, 1, sw1, True)
            wout(j, 0, sw0, False)

            @pl.when(j + 2 < _RPG)
            def _():
                gath(j + 2, 0, sg0, True)

            wout(j + 1, 1, sw1, False)

            @pl.when(j + 3 < _RPG)
            def _():
                gath(j + 3, 1, sg1, True)

    return gather_k(tabs[0], tabs[1], tabs[2], tabs[3], cell2d, iota2d)


def kernel(coords, hidden_state, cell_state, W_ih, W_hh, b_ih, b_hh):
    wih_t = W_ih.T.astype(jnp.bfloat16)
    whh_t = W_hh.T.astype(jnp.bfloat16)
    b2 = (b_ih + b_hh)[None, :]
    coords_t = coords.T.astype(jnp.bfloat16)
    xs2d = coords[:, 0].reshape(_NROW, HS)
    ys2d = coords[:, 1].reshape(_NROW, HS)
    h0, c_v1, cd0 = _lstm_tc(0, coords_t, hidden_state, cell_state,
                             xs2d, ys2d, wih_t, whh_t, b2)
    h1, c_new, cd1 = _lstm_tc(1, coords_t, hidden_state, cell_state,
                              xs2d, ys2d, wih_t, whh_t, b2, c_donate=c_v1)
    t00, t01 = _sc_scatter(h0, cd0)
    t10, t11 = _sc_scatter(h1, cd1)
    cell2d = jnp.concatenate([cd0, cd1], axis=0)
    iota2d = jnp.arange(NCELL, dtype=jnp.int32).reshape(NCELL // HS, HS)
    h_social = _sc_gather((t00, t01, t10, t11), cell2d, iota2d)
    return (h_social, c_new)


# speedup vs baseline: 4.1467x; 1.0656x over previous
import functools
import jax
import jax.numpy as jnp
from jax import lax
from jax.experimental import pallas as pl
from jax.experimental.pallas import tpu as pltpu
from jax.experimental.pallas import tpu_sc as plsc

N = 65536
HS = 128
NG = 64
NCELL = NG * NG
X_MIN, X_MAX = -3.0, 3.0
Y_MIN, Y_MAX = -3.0, 3.0
DX = (X_MAX - X_MIN) / NG
DY = (Y_MAX - Y_MIN) / NG

_TC_B = 1024
_NROW = N // HS
_RB = _TC_B // HS


def _lstm_tc_body(x_ref, h_ref, c_ref, xs_ref, ys_ref, wih_ref, whh_ref,
                  b_ref, hnew_ref, cnew_ref, cell_ref):
    xt = x_ref[...]
    h = h_ref[...].astype(jnp.bfloat16)
    c = c_ref[...]
    gates = (lax.dot_general(xt, wih_ref[...], (((0,), (0,)), ((), ())),
                             preferred_element_type=jnp.float32)
             + jnp.dot(h, whh_ref[...], preferred_element_type=jnp.float32)
             + b_ref[...])

    def sigmoid(z):
        return 0.5 * jnp.tanh(0.5 * z) + 0.5

    i = sigmoid(gates[:, 0:HS])
    f = sigmoid(gates[:, HS:2 * HS])
    g = jnp.tanh(gates[:, 2 * HS:3 * HS])
    o = sigmoid(gates[:, 3 * HS:4 * HS])
    c_new = f * c + i * g
    hnew_ref[...] = o * jnp.tanh(c_new)
    cnew_ref[...] = c_new
    xc = jnp.clip(xs_ref[...], X_MIN, X_MAX)
    yc = jnp.clip(ys_ref[...], Y_MIN, Y_MAX)
    xi = jnp.clip(jnp.floor((xc - X_MIN) / DX).astype(jnp.int32), 0, NG - 1)
    yi = jnp.clip(jnp.floor((yc - Y_MIN) / DY).astype(jnp.int32), 0, NG - 1)
    cell_ref[...] = xi * NG + yi


_NCHUNK = 2
_CB = N // _NCHUNK // _TC_B
_CROW = _NROW // _NCHUNK


def _lstm_tc_body2(x_ref, h_ref, c_ref, xs_ref, ys_ref, wih_ref, whh_ref,
                   b_ref, cdest_ref, hnew_ref, cnew_ref, cell_ref):
    del cdest_ref
    _lstm_tc_body(x_ref, h_ref, c_ref, xs_ref, ys_ref, wih_ref, whh_ref,
                  b_ref, hnew_ref, cnew_ref, cell_ref)


def _lstm_tc(k, coords_t, h, c, xs2d, ys2d, wih_t, whh_t, b2, c_donate=None,
             interpret=False):
    in_specs = [
        pl.BlockSpec((3, _TC_B), lambda i: (0, i + k * _CB)),
        pl.BlockSpec((_TC_B, HS), lambda i: (i + k * _CB, 0)),
        pl.BlockSpec((_TC_B, HS), lambda i: (i + k * _CB, 0)),
        pl.BlockSpec((_RB, HS), lambda i: (i + k * _CB, 0)),
        pl.BlockSpec((_RB, HS), lambda i: (i + k * _CB, 0)),
        pl.BlockSpec((3, 4 * HS), lambda i: (0, 0)),
        pl.BlockSpec((HS, 4 * HS), lambda i: (0, 0)),
        pl.BlockSpec((1, 4 * HS), lambda i: (0, 0)),
    ]
    args = [coords_t, h, c, xs2d, ys2d, wih_t, whh_t, b2]
    if c_donate is None:
        body = _lstm_tc_body
        aliases = {}
    else:
        body = _lstm_tc_body2
        in_specs = in_specs + [pl.BlockSpec((8, HS), lambda i: (0, 0))]
        args = args + [c_donate]
        aliases = {8: 1}
    return pl.pallas_call(
        body,
        grid=(_CB,),
        in_specs=in_specs,
        out_specs=[
            pl.BlockSpec((_TC_B, HS), lambda i: (i, 0)),
            pl.BlockSpec((_TC_B, HS), lambda i: (i + k * _CB, 0)),
            pl.BlockSpec((_RB, HS), lambda i: (i, 0)),
        ],
        out_shape=[
            jax.ShapeDtypeStruct((N // _NCHUNK, HS), jnp.float32),
            jax.ShapeDtypeStruct((N, HS), jnp.float32),
            jax.ShapeDtypeStruct((_CROW, HS), jnp.int32),
        ],
        input_output_aliases=aliases,
        interpret=interpret,
    )(*args)


@functools.cache
def _sc_mesh():
    return plsc.VectorSubcoreMesh(core_axis_name="c", subcore_axis_name="s",
                                  num_cores=2, num_subcores=16)
_RPC = _CROW // 2 // 16
_RPG = _NROW // 32


_NSLOT = 5


def _sc_scatter(h_new, cell2d):
    @functools.partial(
        pl.kernel,
        out_type=[jax.ShapeDtypeStruct((NCELL, HS), jnp.float32),
                  jax.ShapeDtypeStruct((NCELL, HS), jnp.float32)],
        mesh=_sc_mesh(),
        scratch_types=[
            pltpu.VMEM_SHARED((NCELL, HS), jnp.float32),
            pltpu.VMEM((_RPC, HS), jnp.int32),
            pltpu.VMEM((_NSLOT, HS, HS), jnp.float32),
        ] + [pltpu.SemaphoreType.DMA] * _NSLOT,
    )
    def scatter_k(h_hbm, cell_hbm, t0_hbm, t1_hbm, tbl, idx_v, hbuf, *sems):
        c = lax.axis_index("c")
        s = lax.axis_index("s")
        @pl.loop(0, HS)
        def _(r):
            for cb in range(HS // 16):
                hbuf.at[0, r, pl.ds(cb * 16, 16)][...] = jnp.zeros(
                    (16,), jnp.float32)

        pltpu.sync_copy(hbuf.at[0], tbl.at[pl.ds(s * 256, HS)])
        pltpu.sync_copy(hbuf.at[0], tbl.at[pl.ds(s * 256 + HS, HS)])
        plsc.subcore_barrier()
        row0 = c * (_CROW // 2) + s * _RPC
        pltpu.sync_copy(cell_hbm.at[pl.ds(row0, _RPC)], idx_v)

        def load(w, start):
            slot = w % _NSLOT
            args = (h_hbm.at[pl.ds((row0 + w) * HS, HS)], hbuf.at[slot],
                    sems[slot])
            if start:
                pltpu.async_copy(*args)
            else:
                pltpu.make_async_copy(*args).wait()

        def scat(w, start):
            slot = w % _NSLOT
            args = (hbuf.at[slot], tbl.at[idx_v.at[w]], sems[slot])
            if start:
                pltpu.async_copy(*args, add=True)
            else:
                pltpu.make_async_copy(*args).wait()

        for w in range(min(_NSLOT, _RPC)):
            load(w, True)
        for w in range(_RPC):
            load(w, False)
            scat(w, True)
            if w + _NSLOT < _RPC:
                scat(w, False)
                load(w + _NSLOT, True)
        for w in range(max(0, _RPC - _NSLOT), _RPC):
            scat(w, False)

        plsc.subcore_barrier()

        @pl.when(c == 0)
        def _():
            pltpu.sync_copy(tbl.at[pl.ds(s * 256, 256)],
                            t0_hbm.at[pl.ds(s * 256, 256)])

        @pl.when(c == 1)
        def _():
            pltpu.sync_copy(tbl.at[pl.ds(s * 256, 256)],
                            t1_hbm.at[pl.ds(s * 256, 256)])

    return scatter_k(h_new, cell2d)


def _sc_gather(tabs, cell2d, iota2d):
    @functools.partial(
        pl.kernel,
        out_type=jax.ShapeDtypeStruct((N, HS), jnp.float32),
        mesh=_sc_mesh(),
        scratch_types=[
            pltpu.VMEM_SHARED((NCELL, HS), jnp.float32),
            pltpu.VMEM((2, HS), jnp.int32),
            pltpu.VMEM((_RPG, HS), jnp.int32),
            pltpu.VMEM((_NSLOT, HS, HS), jnp.float32),
        ] + [pltpu.SemaphoreType.DMA] * _NSLOT,
    )
    def gather_k(ta_hbm, tb_hbm, tc_hbm, td_hbm, cell_hbm, iota_hbm, o_hbm,
                 tbl, iv, idx_v, buf, *sems):
        c = lax.axis_index("c")
        s = lax.axis_index("s")
        pltpu.sync_copy(iota_hbm.at[pl.ds(2 * s, 2)], iv)
        srcs = [(t.at[pl.ds(s * 256 + h * HS, HS)], h)
                for t in (tb_hbm, tc_hbm, td_hbm) for h in range(2)]
        np_ = len(srcs)
        for p in range(min(_NSLOT, np_)):
            pltpu.async_copy(srcs[p][0], buf.at[p], sems[p])
        pltpu.sync_copy(ta_hbm.at[pl.ds(s * 256, 256)],
                        tbl.at[pl.ds(s * 256, 256)])
        for p, (src, h) in enumerate(srcs):
            slot = p % _NSLOT
            pltpu.make_async_copy(src, buf.at[slot], sems[slot]).wait()
            pltpu.async_copy(buf.at[slot], tbl.at[iv.at[h]], sems[slot],
                             add=True)
            if p + _NSLOT < np_:
                pn = p + _NSLOT
                pltpu.make_async_copy(buf.at[slot], tbl.at[iv.at[h]],
                                      sems[slot]).wait()
                pltpu.async_copy(srcs[pn][0], buf.at[slot], sems[slot])
        for p in range(max(0, np_ - _NSLOT), np_):
            slot = p % _NSLOT
            pltpu.make_async_copy(buf.at[slot], tbl.at[iv.at[srcs[p][1]]],
                                  sems[slot]).wait()
        plsc.subcore_barrier()
        row0 = (c * 16 + s) * _RPG
        pltpu.sync_copy(cell_hbm.at[pl.ds(row0, _RPG)], idx_v)

        def gath(w, start):
            slot = w % _NSLOT
            args = (tbl.at[idx_v.at[w]], buf.at[slot], sems[slot])
            if start:
                pltpu.async_copy(*args)
            else:
                pltpu.make_async_copy(*args).wait()

        def wout(w, start):
            slot = w % _NSLOT
            args = (buf.at[slot], o_hbm.at[pl.ds((row0 + w) * HS, HS)],
                    sems[slot])
            if start:
                pltpu.async_copy(*args)
            else:
                pltpu.make_async_copy(*args).wait()

        for w in range(min(_NSLOT, _RPG)):
            gath(w, True)
        for w in range(_RPG):
            gath(w, False)
            wout(w, True)
            if w + _NSLOT < _RPG:
                wout(w, False)
                gath(w + _NSLOT, True)
        for w in range(max(0, _RPG - _NSLOT), _RPG):
            wout(w, False)

    return gather_k(tabs[0], tabs[1], tabs[2], tabs[3], cell2d, iota2d)


def kernel(coords, hidden_state, cell_state, W_ih, W_hh, b_ih, b_hh):
    wih_t = W_ih.T.astype(jnp.bfloat16)
    whh_t = W_hh.T.astype(jnp.bfloat16)
    b2 = (b_ih + b_hh)[None, :]
    coords_t = coords.T.astype(jnp.bfloat16)
    xs2d = coords[:, 0].reshape(_NROW, HS)
    ys2d = coords[:, 1].reshape(_NROW, HS)
    h0, c_v1, cd0 = _lstm_tc(0, coords_t, hidden_state, cell_state,
                             xs2d, ys2d, wih_t, whh_t, b2)
    h1, c_new, cd1 = _lstm_tc(1, coords_t, hidden_state, cell_state,
                              xs2d, ys2d, wih_t, whh_t, b2, c_donate=c_v1)
    iota2d = jnp.arange(NCELL, dtype=jnp.int32).reshape(NCELL // HS, HS)
    t00, t01 = _sc_scatter(h0, cd0)
    t10, t11 = _sc_scatter(h1, cd1)
    cell2d = jnp.concatenate([cd0, cd1], axis=0)
    h_social = _sc_gather((t00, t01, t10, t11), cell2d, iota2d)
    return (h_social, c_new)


# speedup vs baseline: 4.2520x; 1.0254x over previous
import functools
import jax
import jax.numpy as jnp
from jax import lax
from jax.experimental import pallas as pl
from jax.experimental.pallas import tpu as pltpu
from jax.experimental.pallas import tpu_sc as plsc

N = 65536
HS = 128
NG = 64
NCELL = NG * NG
X_MIN, X_MAX = -3.0, 3.0
Y_MIN, Y_MAX = -3.0, 3.0
DX = (X_MAX - X_MIN) / NG
DY = (Y_MAX - Y_MIN) / NG

_TC_B = 1024
_NROW = N // HS
_RB = _TC_B // HS


def _lstm_tc_body(x_ref, h_ref, c_ref, xs_ref, ys_ref, wih_ref, whh_ref,
                  b_ref, hnew_ref, cnew_ref, cell_ref):
    xt = x_ref[...]
    h = h_ref[...].astype(jnp.bfloat16)
    c = c_ref[...]
    gates = (lax.dot_general(xt, wih_ref[...], (((0,), (0,)), ((), ())),
                             preferred_element_type=jnp.float32)
             + jnp.dot(h, whh_ref[...], preferred_element_type=jnp.float32)
             + b_ref[...])

    def sigmoid(z):
        return 0.5 * jnp.tanh(0.5 * z) + 0.5

    i = sigmoid(gates[:, 0:HS])
    f = sigmoid(gates[:, HS:2 * HS])
    g = jnp.tanh(gates[:, 2 * HS:3 * HS])
    o = sigmoid(gates[:, 3 * HS:4 * HS])
    c_new = f * c + i * g
    hnew_ref[...] = o * jnp.tanh(c_new)
    cnew_ref[...] = c_new
    xc = jnp.clip(xs_ref[...], X_MIN, X_MAX)
    yc = jnp.clip(ys_ref[...], Y_MIN, Y_MAX)
    xi = jnp.clip(jnp.floor((xc - X_MIN) / DX).astype(jnp.int32), 0, NG - 1)
    yi = jnp.clip(jnp.floor((yc - Y_MIN) / DY).astype(jnp.int32), 0, NG - 1)
    cell_ref[...] = xi * NG + yi


_NCHUNK = 2
_CB = N // _NCHUNK // _TC_B
_CROW = _NROW // _NCHUNK


def _lstm_tc_body2(x_ref, h_ref, c_ref, xs_ref, ys_ref, wih_ref, whh_ref,
                   b_ref, cdest_ref, hnew_ref, cnew_ref, cell_ref):
    del cdest_ref
    _lstm_tc_body(x_ref, h_ref, c_ref, xs_ref, ys_ref, wih_ref, whh_ref,
                  b_ref, hnew_ref, cnew_ref, cell_ref)


def _lstm_tc(k, coords_t, h, c, xs2d, ys2d, wih_t, whh_t, b2, c_donate=None,
             interpret=False):
    in_specs = [
        pl.BlockSpec((3, _TC_B), lambda i: (0, i + k * _CB)),
        pl.BlockSpec((_TC_B, HS), lambda i: (i + k * _CB, 0)),
        pl.BlockSpec((_TC_B, HS), lambda i: (i + k * _CB, 0)),
        pl.BlockSpec((_RB, HS), lambda i: (i + k * _CB, 0)),
        pl.BlockSpec((_RB, HS), lambda i: (i + k * _CB, 0)),
        pl.BlockSpec((3, 4 * HS), lambda i: (0, 0)),
        pl.BlockSpec((HS, 4 * HS), lambda i: (0, 0)),
        pl.BlockSpec((1, 4 * HS), lambda i: (0, 0)),
    ]
    args = [coords_t, h, c, xs2d, ys2d, wih_t, whh_t, b2]
    if c_donate is None:
        body = _lstm_tc_body
        aliases = {}
    else:
        body = _lstm_tc_body2
        in_specs = in_specs + [pl.BlockSpec((8, HS), lambda i: (0, 0))]
        args = args + [c_donate]
        aliases = {8: 1}
    return pl.pallas_call(
        body,
        grid=(_CB,),
        in_specs=in_specs,
        out_specs=[
            pl.BlockSpec((_TC_B, HS), lambda i: (i, 0)),
            pl.BlockSpec((_TC_B, HS), lambda i: (i + k * _CB, 0)),
            pl.BlockSpec((_RB, HS), lambda i: (i, 0)),
        ],
        out_shape=[
            jax.ShapeDtypeStruct((N // _NCHUNK, HS), jnp.float32),
            jax.ShapeDtypeStruct((N, HS), jnp.float32),
            jax.ShapeDtypeStruct((_CROW, HS), jnp.int32),
        ],
        input_output_aliases=aliases,
        interpret=interpret,
    )(*args)


@functools.cache
def _sc_mesh():
    return plsc.VectorSubcoreMesh(core_axis_name="c", subcore_axis_name="s",
                                  num_cores=2, num_subcores=16)
_RPC = _CROW // 2 // 16
_RPG = _NROW // 32


_NSLOT = 5


def _sc_scatter(h_new, cell2d, inits=None):
    init_args = [] if inits is None else list(inits)
    @functools.partial(
        pl.kernel,
        out_type=[jax.ShapeDtypeStruct((NCELL, HS), jnp.float32),
                  jax.ShapeDtypeStruct((NCELL, HS), jnp.float32)],
        mesh=_sc_mesh(),
        scratch_types=[
            pltpu.VMEM_SHARED((NCELL, HS), jnp.float32),
            pltpu.VMEM((_RPC, HS), jnp.int32),
            pltpu.VMEM((_NSLOT, HS, HS), jnp.float32),
        ] + [pltpu.SemaphoreType.DMA] * _NSLOT,
    )
    def scatter_k(h_hbm, cell_hbm, *rest):
        if inits is None:
            (t0_hbm, t1_hbm, tbl, idx_v, hbuf), sems = rest[:5], rest[5:]
        else:
            (i0_hbm, i1_hbm, t0_hbm, t1_hbm, tbl, idx_v, hbuf) = rest[:7]
            sems = rest[7:]
        c = lax.axis_index("c")
        s = lax.axis_index("s")
        if inits is None:
            @pl.loop(0, HS)
            def _(r):
                for cb in range(HS // 16):
                    hbuf.at[0, r, pl.ds(cb * 16, 16)][...] = jnp.zeros(
                        (16,), jnp.float32)

            pltpu.sync_copy(hbuf.at[0], tbl.at[pl.ds(s * 256, HS)])
            pltpu.sync_copy(hbuf.at[0], tbl.at[pl.ds(s * 256 + HS, HS)])
        else:
            @pl.when(c == 0)
            def _():
                pltpu.sync_copy(i0_hbm.at[pl.ds(s * 256, 256)],
                                tbl.at[pl.ds(s * 256, 256)])

            @pl.when(c == 1)
            def _():
                pltpu.sync_copy(i1_hbm.at[pl.ds(s * 256, 256)],
                                tbl.at[pl.ds(s * 256, 256)])
        plsc.subcore_barrier()
        row0 = c * (_CROW // 2) + s * _RPC
        pltpu.sync_copy(cell_hbm.at[pl.ds(row0, _RPC)], idx_v)

        def load(w, start):
            slot = w % _NSLOT
            args = (h_hbm.at[pl.ds((row0 + w) * HS, HS)], hbuf.at[slot],
                    sems[slot])
            if start:
                pltpu.async_copy(*args)
            else:
                pltpu.make_async_copy(*args).wait()

        def scat(w, start):
            slot = w % _NSLOT
            args = (hbuf.at[slot], tbl.at[idx_v.at[w]], sems[slot])
            if start:
                pltpu.async_copy(*args, add=True)
            else:
                pltpu.make_async_copy(*args).wait()

        for w in range(min(_NSLOT, _RPC)):
            load(w, True)
        for w in range(_RPC):
            load(w, False)
            scat(w, True)
            if w + _NSLOT < _RPC:
                scat(w, False)
                load(w + _NSLOT, True)
        for w in range(max(0, _RPC - _NSLOT), _RPC):
            scat(w, False)

        plsc.subcore_barrier()

        @pl.when(c == 0)
        def _():
            pltpu.sync_copy(tbl.at[pl.ds(s * 256, 256)],
                            t0_hbm.at[pl.ds(s * 256, 256)])

        @pl.when(c == 1)
        def _():
            pltpu.sync_copy(tbl.at[pl.ds(s * 256, 256)],
                            t1_hbm.at[pl.ds(s * 256, 256)])

    return scatter_k(h_new, cell2d, *init_args)


def _sc_gather(tabs, cell2d, iota2d):
    @functools.partial(
        pl.kernel,
        out_type=jax.ShapeDtypeStruct((N, HS), jnp.float32),
        mesh=_sc_mesh(),
        scratch_types=[
            pltpu.VMEM_SHARED((NCELL, HS), jnp.float32),
            pltpu.VMEM((2, HS), jnp.int32),
            pltpu.VMEM((_RPG, HS), jnp.int32),
            pltpu.VMEM((_NSLOT, HS, HS), jnp.float32),
        ] + [pltpu.SemaphoreType.DMA] * _NSLOT,
    )
    def gather_k(ta_hbm, tb_hbm, cell_hbm, iota_hbm, o_hbm,
                 tbl, iv, idx_v, buf, *sems):
        c = lax.axis_index("c")
        s = lax.axis_index("s")
        pltpu.sync_copy(iota_hbm.at[pl.ds(2 * s, 2)], iv)

        def stage(own_hbm, other_hbm):
            for h in range(2):
                pltpu.async_copy(
                    other_hbm.at[pl.ds(s * 256 + h * HS, HS)], buf.at[h],
                    sems[h])
            pltpu.sync_copy(own_hbm.at[pl.ds(s * 256, 256)],
                            tbl.at[pl.ds(s * 256, 256)])
            for h in range(2):
                pltpu.make_async_copy(
                    other_hbm.at[pl.ds(s * 256 + h * HS, HS)], buf.at[h],
                    sems[h]).wait()
                pltpu.async_copy(buf.at[h], tbl.at[iv.at[h]], sems[h],
                                 add=True)
            for h in range(2):
                pltpu.make_async_copy(buf.at[h], tbl.at[iv.at[h]],
                                      sems[h]).wait()

        @pl.when(c == 0)
        def _():
            stage(ta_hbm, tb_hbm)

        @pl.when(c == 1)
        def _():
            stage(tb_hbm, ta_hbm)
        plsc.subcore_barrier()
        row0 = (c * 16 + s) * _RPG
        pltpu.sync_copy(cell_hbm.at[pl.ds(row0, _RPG)], idx_v)

        def gath(w, start):
            slot = w % _NSLOT
            args = (tbl.at[idx_v.at[w]], buf.at[slot], sems[slot])
            if start:
                pltpu.async_copy(*args)
            else:
                pltpu.make_async_copy(*args).wait()

        def wout(w, start):
            slot = w % _NSLOT
            args = (buf.at[slot], o_hbm.at[pl.ds((row0 + w) * HS, HS)],
                    sems[slot])
            if start:
                pltpu.async_copy(*args)
            else:
                pltpu.make_async_copy(*args).wait()

        for w in range(min(_NSLOT, _RPG)):
            gath(w, True)
        for w in range(_RPG):
            gath(w, False)
            wout(w, True)
            if w + _NSLOT < _RPG:
                wout(w, False)
                gath(w + _NSLOT, True)
        for w in range(max(0, _RPG - _NSLOT), _RPG):
            wout(w, False)

    return gather_k(tabs[0], tabs[1], cell2d, iota2d)


def kernel(coords, hidden_state, cell_state, W_ih, W_hh, b_ih, b_hh):
    wih_t = W_ih.T.astype(jnp.bfloat16)
    whh_t = W_hh.T.astype(jnp.bfloat16)
    b2 = (b_ih + b_hh)[None, :]
    coords_t = coords.T.astype(jnp.bfloat16)
    xs2d = coords[:, 0].reshape(_NROW, HS)
    ys2d = coords[:, 1].reshape(_NROW, HS)
    h0, c_v1, cd0 = _lstm_tc(0, coords_t, hidden_state, cell_state,
                             xs2d, ys2d, wih_t, whh_t, b2)
    h1, c_new, cd1 = _lstm_tc(1, coords_t, hidden_state, cell_state,
                              xs2d, ys2d, wih_t, whh_t, b2, c_donate=c_v1)
    iota2d = jnp.arange(NCELL, dtype=jnp.int32).reshape(NCELL // HS, HS)
    t00, t01 = _sc_scatter(h0, cd0)
    t10, t11 = _sc_scatter(h1, cd1, inits=(t00, t01))
    cell2d = jnp.concatenate([cd0, cd1], axis=0)
    h_social = _sc_gather((t10, t11), cell2d, iota2d)
    return (h_social, c_new)


# speedup vs baseline: 4.6724x; 1.0989x over previous
import functools
import jax
import jax.numpy as jnp
from jax import lax
from jax.experimental import pallas as pl
from jax.experimental.pallas import tpu as pltpu
from jax.experimental.pallas import tpu_sc as plsc

N = 65536
HS = 128
NG = 64
NCELL = NG * NG
X_MIN, X_MAX = -3.0, 3.0
Y_MIN, Y_MAX = -3.0, 3.0
DX = (X_MAX - X_MIN) / NG
DY = (Y_MAX - Y_MIN) / NG

_TC_B = 2048
_NROW = N // HS
_RB = _TC_B // HS


def _lstm_tc_body(x_ref, h_ref, c_ref, xs_ref, ys_ref, wih_ref, whh_ref,
                  b_ref, hnew_ref, cnew_ref, cell_ref):
    xt = x_ref[...]
    h = h_ref[...].astype(jnp.bfloat16)
    c = c_ref[...]
    gates = (lax.dot_general(xt, wih_ref[...], (((0,), (0,)), ((), ())),
                             preferred_element_type=jnp.float32)
             + jnp.dot(h, whh_ref[...], preferred_element_type=jnp.float32)
             + b_ref[...])

    def sigmoid(z):
        return 0.5 * jnp.tanh(0.5 * z) + 0.5

    i = sigmoid(gates[:, 0:HS])
    f = sigmoid(gates[:, HS:2 * HS])
    g = jnp.tanh(gates[:, 2 * HS:3 * HS])
    o = sigmoid(gates[:, 3 * HS:4 * HS])
    c_new = f * c + i * g
    hnew_ref[...] = o * jnp.tanh(c_new)
    cnew_ref[...] = c_new
    xc = jnp.clip(xs_ref[...], X_MIN, X_MAX)
    yc = jnp.clip(ys_ref[...], Y_MIN, Y_MAX)
    xi = jnp.clip(jnp.floor((xc - X_MIN) / DX).astype(jnp.int32), 0, NG - 1)
    yi = jnp.clip(jnp.floor((yc - Y_MIN) / DY).astype(jnp.int32), 0, NG - 1)
    cell_ref[...] = xi * NG + yi


_NCHUNK = 2
_CB = N // _NCHUNK // _TC_B
_CROW = _NROW // _NCHUNK


def _lstm_tc_body2(x_ref, h_ref, c_ref, xs_ref, ys_ref, wih_ref, whh_ref,
                   b_ref, cdest_ref, hnew_ref, cnew_ref, cell_ref):
    del cdest_ref
    _lstm_tc_body(x_ref, h_ref, c_ref, xs_ref, ys_ref, wih_ref, whh_ref,
                  b_ref, hnew_ref, cnew_ref, cell_ref)


def _lstm_tc(k, coords_t, h, c, xs2d, ys2d, wih_t, whh_t, b2, c_donate=None,
             interpret=False):
    in_specs = [
        pl.BlockSpec((3, _TC_B), lambda i: (0, i + k * _CB)),
        pl.BlockSpec((_TC_B, HS), lambda i: (i + k * _CB, 0)),
        pl.BlockSpec((_TC_B, HS), lambda i: (i + k * _CB, 0)),
        pl.BlockSpec((_RB, HS), lambda i: (i + k * _CB, 0)),
        pl.BlockSpec((_RB, HS), lambda i: (i + k * _CB, 0)),
        pl.BlockSpec((3, 4 * HS), lambda i: (0, 0)),
        pl.BlockSpec((HS, 4 * HS), lambda i: (0, 0)),
        pl.BlockSpec((1, 4 * HS), lambda i: (0, 0)),
    ]
    args = [coords_t, h, c, xs2d, ys2d, wih_t, whh_t, b2]
    if c_donate is None:
        body = _lstm_tc_body
        aliases = {}
    else:
        body = _lstm_tc_body2
        in_specs = in_specs + [pl.BlockSpec((8, HS), lambda i: (0, 0))]
        args = args + [c_donate]
        aliases = {8: 1}
    return pl.pallas_call(
        body,
        grid=(_CB,),
        in_specs=in_specs,
        out_specs=[
            pl.BlockSpec((_TC_B, HS), lambda i: (i, 0)),
            pl.BlockSpec((_TC_B, HS), lambda i: (i + k * _CB, 0)),
            pl.BlockSpec((_RB, HS), lambda i: (i, 0)),
        ],
        out_shape=[
            jax.ShapeDtypeStruct((N // _NCHUNK, HS), jnp.float32),
            jax.ShapeDtypeStruct((N, HS), jnp.float32),
            jax.ShapeDtypeStruct((_CROW, HS), jnp.int32),
        ],
        input_output_aliases=aliases,
        interpret=interpret,
    )(*args)


@functools.cache
def _sc_mesh():
    return plsc.VectorSubcoreMesh(core_axis_name="c", subcore_axis_name="s",
                                  num_cores=2, num_subcores=16)
_RPC = _CROW // 2 // 16
_RPG = _NROW // 32


_NSLOT = 5


def _sc_scatter(h_new, cell2d, inits=None):
    init_args = [] if inits is None else list(inits)
    @functools.partial(
        pl.kernel,
        out_type=[jax.ShapeDtypeStruct((NCELL, HS), jnp.float32),
                  jax.ShapeDtypeStruct((NCELL, HS), jnp.float32)],
        mesh=_sc_mesh(),
        scratch_types=[
            pltpu.VMEM_SHARED((NCELL, HS), jnp.float32),
            pltpu.VMEM((_RPC, HS), jnp.int32),
            pltpu.VMEM((_NSLOT, HS, HS), jnp.float32),
        ] + [pltpu.SemaphoreType.DMA] * _NSLOT,
    )
    def scatter_k(h_hbm, cell_hbm, *rest):
        if inits is None:
            (t0_hbm, t1_hbm, tbl, idx_v, hbuf), sems = rest[:5], rest[5:]
        else:
            (i0_hbm, i1_hbm, t0_hbm, t1_hbm, tbl, idx_v, hbuf) = rest[:7]
            sems = rest[7:]
        c = lax.axis_index("c")
        s = lax.axis_index("s")
        if inits is None:
            @pl.loop(0, HS)
            def _(r):
                for cb in range(HS // 16):
                    hbuf.at[0, r, pl.ds(cb * 16, 16)][...] = jnp.zeros(
                        (16,), jnp.float32)

            pltpu.sync_copy(hbuf.at[0], tbl.at[pl.ds(s * 256, HS)])
            pltpu.sync_copy(hbuf.at[0], tbl.at[pl.ds(s * 256 + HS, HS)])
        else:
            @pl.when(c == 0)
            def _():
                pltpu.sync_copy(i0_hbm.at[pl.ds(s * 256, 256)],
                                tbl.at[pl.ds(s * 256, 256)])

            @pl.when(c == 1)
            def _():
                pltpu.sync_copy(i1_hbm.at[pl.ds(s * 256, 256)],
                                tbl.at[pl.ds(s * 256, 256)])
        plsc.subcore_barrier()
        row0 = c * (_CROW // 2) + s * _RPC
        pltpu.sync_copy(cell_hbm.at[pl.ds(row0, _RPC)], idx_v)

        def load(w, start):
            slot = w % _NSLOT
            args = (h_hbm.at[pl.ds((row0 + w) * HS, HS)], hbuf.at[slot],
                    sems[slot])
            if start:
                pltpu.async_copy(*args)
            else:
                pltpu.make_async_copy(*args).wait()

        def scat(w, start):
            slot = w % _NSLOT
            args = (hbuf.at[slot], tbl.at[idx_v.at[w]], sems[slot])
            if start:
                pltpu.async_copy(*args, add=True)
            else:
                pltpu.make_async_copy(*args).wait()

        for w in range(min(_NSLOT, _RPC)):
            load(w, True)
        for w in range(_RPC):
            load(w, False)
            scat(w, True)
            if w + _NSLOT < _RPC:
                scat(w, False)
                load(w + _NSLOT, True)
        for w in range(max(0, _RPC - _NSLOT), _RPC):
            scat(w, False)

        plsc.subcore_barrier()

        @pl.when(c == 0)
        def _():
            pltpu.sync_copy(tbl.at[pl.ds(s * 256, 256)],
                            t0_hbm.at[pl.ds(s * 256, 256)])

        @pl.when(c == 1)
        def _():
            pltpu.sync_copy(tbl.at[pl.ds(s * 256, 256)],
                            t1_hbm.at[pl.ds(s * 256, 256)])

    return scatter_k(h_new, cell2d, *init_args)


def _sc_gather(tabs, cell2d, iota2d):
    @functools.partial(
        pl.kernel,
        out_type=jax.ShapeDtypeStruct((N, HS), jnp.float32),
        mesh=_sc_mesh(),
        scratch_types=[
            pltpu.VMEM_SHARED((NCELL, HS), jnp.float32),
            pltpu.VMEM((2, HS), jnp.int32),
            pltpu.VMEM((_RPG, HS), jnp.int32),
            pltpu.VMEM((_NSLOT, HS, HS), jnp.float32),
        ] + [pltpu.SemaphoreType.DMA] * _NSLOT,
    )
    def gather_k(ta_hbm, tb_hbm, cell_hbm, iota_hbm, o_hbm,
                 tbl, iv, idx_v, buf, *sems):
        c = lax.axis_index("c")
        s = lax.axis_index("s")
        pltpu.sync_copy(iota_hbm.at[pl.ds(2 * s, 2)], iv)

        def stage(own_hbm, other_hbm):
            for h in range(2):
                pltpu.async_copy(
                    other_hbm.at[pl.ds(s * 256 + h * HS, HS)], buf.at[h],
                    sems[h])
            pltpu.sync_copy(own_hbm.at[pl.ds(s * 256, 256)],
                            tbl.at[pl.ds(s * 256, 256)])
            for h in range(2):
                pltpu.make_async_copy(
                    other_hbm.at[pl.ds(s * 256 + h * HS, HS)], buf.at[h],
                    sems[h]).wait()
                pltpu.async_copy(buf.at[h], tbl.at[iv.at[h]], sems[h],
                                 add=True)
            for h in range(2):
                pltpu.make_async_copy(buf.at[h], tbl.at[iv.at[h]],
                                      sems[h]).wait()

        @pl.when(c == 0)
        def _():
            stage(ta_hbm, tb_hbm)

        @pl.when(c == 1)
        def _():
            stage(tb_hbm, ta_hbm)
        plsc.subcore_barrier()
        row0 = (c * 16 + s) * _RPG
        pltpu.sync_copy(cell_hbm.at[pl.ds(row0, _RPG)], idx_v)

        def gath(w, start):
            slot = w % _NSLOT
            args = (tbl.at[idx_v.at[w]], buf.at[slot], sems[slot])
            if start:
                pltpu.async_copy(*args)
            else:
                pltpu.make_async_copy(*args).wait()

        def wout(w, start):
            slot = w % _NSLOT
            args = (buf.at[slot], o_hbm.at[pl.ds((row0 + w) * HS, HS)],
                    sems[slot])
            if start:
                pltpu.async_copy(*args)
            else:
                pltpu.make_async_copy(*args).wait()

        for w in range(min(_NSLOT, _RPG)):
            gath(w, True)
        for w in range(_RPG):
            gath(w, False)
            wout(w, True)
            if w + _NSLOT < _RPG:
                wout(w, False)
                gath(w + _NSLOT, True)
        for w in range(max(0, _RPG - _NSLOT), _RPG):
            wout(w, False)

    return gather_k(tabs[0], tabs[1], cell2d, iota2d)


def kernel(coords, hidden_state, cell_state, W_ih, W_hh, b_ih, b_hh):
    wih_t = W_ih.T.astype(jnp.bfloat16)
    whh_t = W_hh.T.astype(jnp.bfloat16)
    b2 = (b_ih + b_hh)[None, :]
    coords_t = coords.T.astype(jnp.bfloat16)
    xs2d = coords[:, 0].reshape(_NROW, HS)
    ys2d = coords[:, 1].reshape(_NROW, HS)
    h0, c_v1, cd0 = _lstm_tc(0, coords_t, hidden_state, cell_state,
                             xs2d, ys2d, wih_t, whh_t, b2)
    h1, c_new, cd1 = _lstm_tc(1, coords_t, hidden_state, cell_state,
                              xs2d, ys2d, wih_t, whh_t, b2, c_donate=c_v1)
    iota2d = jnp.arange(NCELL, dtype=jnp.int32).reshape(NCELL // HS, HS)
    t00, t01 = _sc_scatter(h0, cd0)
    t10, t11 = _sc_scatter(h1, cd1, inits=(t00, t01))
    cell2d = jnp.concatenate([cd0, cd1], axis=0)
    h_social = _sc_gather((t10, t11), cell2d, iota2d)
    return (h_social, c_new)
